# baseline (speedup 1.0000x reference)
import jax
import jax.numpy as jnp
from jax import lax
from jax.experimental import pallas as pl
from jax.experimental.pallas import tpu as pltpu
from jax.experimental.pallas import tpu_sc as plsc

N = 10000
NP = 10240
GARB = 10016
D = 128
DE = 16
E = 320000
EP = 327680
CH = 128
CPT = 80
IST = 8
NC, NS = 2, 16
RPS = NP // NS
NBLK = 512
NGRID = NP // NBLK
NG = 64


def _mesh():
    return plsc.VectorSubcoreMesh(core_axis_name="c", subcore_axis_name="s")


def _seg_t_kernel(t_hbm, dst_hbm, z128_hbm, one128_hbm,
                  outT_hbm, outC_hbm,
                  dsti, rows0, rows1, acc, sg0, sg1, ss0, ss1):
    c = lax.axis_index("c")
    s = lax.axis_index("s")
    w = c * NS + s
    rows = (rows0, rows1)
    sg = (sg0, sg1)
    ss = (ss0, ss1)

    pltpu.sync_copy(z128_hbm, rows0)

    @pl.loop(0, RPS // CH)
    def _(k):
        pltpu.sync_copy(rows0, acc.at[pl.ds(s * RPS + k * CH, CH)])

    plsc.subcore_barrier()

    @pl.loop(0, CPT // IST)
    def _(jo):
        base = w * CPT + jo * IST
        pltpu.sync_copy(dst_hbm.at[pl.ds(base, IST)], dsti)
        gd = [None, None]
        sd = [None, None]
        gd[0] = pltpu.async_copy(t_hbm.at[pl.ds(base * CH, CH)], rows[0], sg[0])
        for jj in range(IST):
            k = jj % 2
            gd[k].wait()
            if jj + 1 < IST:
                k2 = (jj + 1) % 2
                if sd[k2] is not None:
                    sd[k2].wait()
                gd[k2] = pltpu.async_copy(
                    t_hbm.at[pl.ds((base + jj + 1) * CH, CH)], rows[k2], sg[k2])
            sd[k] = pltpu.async_copy(rows[k], acc.at[dsti.at[jj]], ss[k], add=True)
        sd[0].wait()
        sd[1].wait()

    plsc.subcore_barrier()

    @pl.loop(0, RPS // CH)
    def _(k):
        pltpu.sync_copy(acc.at[pl.ds(s * RPS + k * CH, CH)], rows0)
        pltpu.sync_copy(rows0, outT_hbm.at[c, pl.ds(s * RPS + k * CH, CH)])

    pltpu.sync_copy(z128_hbm, rows0)

    @pl.loop(0, RPS // CH)
    def _(k):
        pltpu.sync_copy(rows0, acc.at[pl.ds(s * RPS + k * CH, CH)])

    pltpu.sync_copy(one128_hbm, rows0)
    plsc.subcore_barrier()

    @pl.loop(0, CPT // IST)
    def _(jo):
        pltpu.sync_copy(dst_hbm.at[pl.ds(w * CPT + jo * IST, IST)], dsti)
        sd = []
        for jj in range(IST):
            sd.append(pltpu.async_copy(rows0, acc.at[dsti.at[jj]], ss0, add=True))
        for d in sd:
            d.wait()

    plsc.subcore_barrier()

    @pl.loop(0, RPS // CH)
    def _(k):
        pltpu.sync_copy(acc.at[pl.ds(s * RPS + k * CH, CH)], rows1)
        pltpu.sync_copy(rows1, outC_hbm.at[c, pl.ds(s * RPS + k * CH, CH)])


def _seg_t(t, dst2d, z128, one128):
    f = pl.kernel(
        _seg_t_kernel,
        out_type=(jax.ShapeDtypeStruct((NC, NP, D), jnp.float32),
                  jax.ShapeDtypeStruct((NC, NP, D), jnp.float32)),
        mesh=_mesh(),
        scratch_types=[
            pltpu.VMEM((IST, CH), jnp.int32),
            pltpu.VMEM((CH, D), jnp.float32),
            pltpu.VMEM((CH, D), jnp.float32),
            pltpu.VMEM_SHARED((NP, D), jnp.float32),
            pltpu.SemaphoreType.DMA,
            pltpu.SemaphoreType.DMA,
            pltpu.SemaphoreType.DMA,
            pltpu.SemaphoreType.DMA,
        ],
    )
    return f(t, dst2d, z128, one128)


CPT0 = 40
CPT1 = 2 * CPT - CPT0


def _seg_h_kernel(h_hbm, src_hbm, dst_hbm, z128_hbm,
                  out_hbm,
                  srci, dsti, rows0, rows1, acc, sg0, sg1, ss0, ss1):
    c = lax.axis_index("c")
    s = lax.axis_index("s")
    rows = (rows0, rows1)
    sg = (sg0, sg1)
    ss = (ss0, ss1)
    pltpu.sync_copy(z128_hbm, rows0)

    @pl.loop(0, RPS // CH)
    def _(k):
        pltpu.sync_copy(rows0, acc.at[pl.ds(s * RPS + k * CH, CH)])

    plsc.subcore_barrier()

    my_cpt = jnp.where(c == 0, CPT0, CPT1)
    my_base = jnp.where(c == 0, s * CPT0, NS * CPT0 + s * CPT1)

    @pl.loop(0, my_cpt // IST)
    def _(jo):
        pltpu.sync_copy(src_hbm.at[pl.ds(my_base + jo * IST, IST)], srci)
        pltpu.sync_copy(dst_hbm.at[pl.ds(my_base + jo * IST, IST)], dsti)
        gd = [None, None]
        sd = [None, None]
        gd[0] = pltpu.async_copy(h_hbm.at[srci.at[0]], rows[0], sg[0])
        for jj in range(IST):
            k = jj % 2
            gd[k].wait()
            if jj + 1 < IST:
                k2 = (jj + 1) % 2
                if sd[k2] is not None:
                    sd[k2].wait()
                gd[k2] = pltpu.async_copy(h_hbm.at[srci.at[jj + 1]], rows[k2], sg[k2])
            sd[k] = pltpu.async_copy(rows[k], acc.at[dsti.at[jj]], ss[k], add=True)
        sd[0].wait()
        sd[1].wait()

    plsc.subcore_barrier()

    @pl.loop(0, RPS // CH)
    def _(k):
        pltpu.sync_copy(acc.at[pl.ds(s * RPS + k * CH, CH)], rows0)
        pltpu.sync_copy(rows0, out_hbm.at[c, pl.ds(s * RPS + k * CH, CH)])


def _seg_h(h, src2d, dst2d, z128):
    f = pl.kernel(
        _seg_h_kernel,
        out_type=jax.ShapeDtypeStruct((NC, NP, D), jnp.float32),
        mesh=_mesh(),
        scratch_types=[
            pltpu.VMEM((IST, CH), jnp.int32),
            pltpu.VMEM((IST, CH), jnp.int32),
            pltpu.VMEM((CH, D), jnp.float32),
            pltpu.VMEM((CH, D), jnp.float32),
            pltpu.VMEM_SHARED((NP, D), jnp.float32),
            pltpu.SemaphoreType.DMA,
            pltpu.SemaphoreType.DMA,
            pltpu.SemaphoreType.DMA,
            pltpu.SemaphoreType.DMA,
        ],
    )
    return f(h, src2d, dst2d, z128)


def _node_mlp_kernel(x_ref, w1_ref, b1_ref, w2_ref, b2_ref, o_ref):
    a = jnp.dot(x_ref[...], w1_ref[...], preferred_element_type=jnp.float32)
    a = jax.nn.relu(a + b1_ref[...])
    o_ref[...] = jnp.dot(a, w2_ref[...], preferred_element_type=jnp.float32) + b2_ref[...]


def _node_mlp(xp, Wn1, bn1, Wn2, bn2):
    return pl.pallas_call(
        _node_mlp_kernel,
        grid=(NGRID,),
        in_specs=[
            pl.BlockSpec((NBLK, D), lambda i: (i, 0)),
            pl.BlockSpec((D, D), lambda i: (0, 0)),
            pl.BlockSpec((1, D), lambda i: (0, 0)),
            pl.BlockSpec((D, D), lambda i: (0, 0)),
            pl.BlockSpec((1, D), lambda i: (0, 0)),
        ],
        out_specs=pl.BlockSpec((NBLK, D), lambda i: (i, 0)),
        out_shape=jax.ShapeDtypeStruct((NP, D), jnp.float32),
    )(xp, Wn1, bn1, Wn2, bn2)


EBLK = 2048


def _edge_mlp_kernel(ea_ref, w1_ref, b1_ref, o_ref):
    a = jnp.dot(ea_ref[...], w1_ref[...], preferred_element_type=jnp.float32)
    o_ref[...] = jax.nn.relu(a + b1_ref[...])


def _edge_mlp(eap, We1, be1):
    return pl.pallas_call(
        _edge_mlp_kernel,
        grid=(EP // EBLK,),
        in_specs=[
            pl.BlockSpec((EBLK, DE), lambda i: (i, 0)),
            pl.BlockSpec((DE, D), lambda i: (0, 0)),
            pl.BlockSpec((1, D), lambda i: (0, 0)),
        ],
        out_specs=pl.BlockSpec((EBLK, D), lambda i: (i, 0)),
        out_shape=jax.ShapeDtypeStruct((EP, D), jnp.float32),
    )(eap, We1, be1)


def _prep_kernel(t0_ref, t1_ref, c0_ref, c1_ref, we2_ref, be2_ref,
                 wt_ref, bt_ref, g0_ref, g1_ref, g2_ref, rinv_ref):
    tsum = t0_ref[0] + t1_ref[0]
    cnt = (c0_ref[0] + c1_ref[0])[:, 0:1]
    rinv_ref[...] = 1.0 / jnp.maximum(cnt, 1.0)
    for i, g_ref in enumerate((g0_ref, g1_ref, g2_ref)):
        m = jnp.dot(we2_ref[...], wt_ref[i], preferred_element_type=jnp.float32)
        v = jnp.dot(be2_ref[...], wt_ref[i], preferred_element_type=jnp.float32) + bt_ref[i]
        g_ref[...] = jnp.dot(tsum, m, preferred_element_type=jnp.float32) + cnt * v


def _prep(Tp, Cp, We2, be2, Wts, bts):
    return pl.pallas_call(
        _prep_kernel,
        grid=(NGRID,),
        in_specs=[
            pl.BlockSpec((1, NBLK, D), lambda i: (0, i, 0)),
            pl.BlockSpec((1, NBLK, D), lambda i: (1, i, 0)),
            pl.BlockSpec((1, NBLK, D), lambda i: (0, i, 0)),
            pl.BlockSpec((1, NBLK, D), lambda i: (1, i, 0)),
            pl.BlockSpec((D, D), lambda i: (0, 0)),
            pl.BlockSpec((1, D), lambda i: (0, 0)),
            pl.BlockSpec((3, D, D), lambda i: (0, 0, 0)),
            pl.BlockSpec((3, 1, D), lambda i: (0, 0, 0)),
        ],
        out_specs=[
            pl.BlockSpec((NBLK, D), lambda i: (i, 0)),
            pl.BlockSpec((NBLK, D), lambda i: (i, 0)),
            pl.BlockSpec((NBLK, D), lambda i: (i, 0)),
            pl.BlockSpec((NBLK, 1), lambda i: (i, 0)),
        ],
        out_shape=[
            jax.ShapeDtypeStruct((NP, D), jnp.float32),
            jax.ShapeDtypeStruct((NP, D), jnp.float32),
            jax.ShapeDtypeStruct((NP, D), jnp.float32),
            jax.ShapeDtypeStruct((NP, 1), jnp.float32),
        ],
    )(Tp, Tp, Cp, Cp, We2, be2, Wts, bts)


def _conv_kernel(p0_ref, p1_ref, g_ref, rinv_ref, h_ref, wl_ref, bl_ref,
                 wr_ref, o_ref):
    s = p0_ref[0] + p1_ref[0]
    agg = (s + g_ref[...]) * rinv_ref[...]
    out = (jnp.dot(agg, wl_ref[...], preferred_element_type=jnp.float32)
           + bl_ref[...]
           + jnp.dot(h_ref[...], wr_ref[...], preferred_element_type=jnp.float32))
    o_ref[...] = jax.nn.relu(h_ref[...] + out)


def _conv(P, G, rinv, h, Wl, bl, Wr):
    return pl.pallas_call(
        _conv_kernel,
        grid=(NGRID,),
        in_specs=[
            pl.BlockSpec((1, NBLK, D), lambda i: (0, i, 0)),
            pl.BlockSpec((1, NBLK, D), lambda i: (1, i, 0)),
            pl.BlockSpec((NBLK, D), lambda i: (i, 0)),
            pl.BlockSpec((NBLK, 1), lambda i: (i, 0)),
            pl.BlockSpec((NBLK, D), lambda i: (i, 0)),
            pl.BlockSpec((D, D), lambda i: (0, 0)),
            pl.BlockSpec((1, D), lambda i: (0, 0)),
            pl.BlockSpec((D, D), lambda i: (0, 0)),
        ],
        out_specs=pl.BlockSpec((NBLK, D), lambda i: (i, 0)),
        out_shape=jax.ShapeDtypeStruct((NP, D), jnp.float32),
    )(P, P, G, rinv, h, Wl, bl, Wr)


def _conv2_pool_kernel(p0_ref, p1_ref, g_ref, rinv_ref, h_ref, wl_ref, bl_ref,
                       wr_ref, batch_ref, o_ref, accs, accc):
    i = pl.program_id(0)
    s = p0_ref[0] + p1_ref[0]
    agg = (s + g_ref[...]) * rinv_ref[...]
    out = (jnp.dot(agg, wl_ref[...], preferred_element_type=jnp.float32)
           + bl_ref[...]
           + jnp.dot(h_ref[...], wr_ref[...], preferred_element_type=jnp.float32))
    gid = lax.broadcasted_iota(jnp.int32, (NBLK, NG), 1)
    oh = (batch_ref[...] == gid).astype(jnp.float32)
    dn = (((0,), (0,)), ((), ()))
    psum = lax.dot_general(oh, out, dn, preferred_element_type=jnp.float32)
    pcnt = lax.dot_general(oh, jnp.ones((NBLK, D), jnp.float32), dn,
                           preferred_element_type=jnp.float32)

    @pl.when(i == 0)
    def _():
        accs[...] = psum
        accc[...] = pcnt

    @pl.when(i > 0)
    def _():
        accs[...] += psum
        accc[...] += pcnt

    @pl.when(i == NGRID - 1)
    def _():
        o_ref[...] = accs[...] / jnp.maximum(accc[...], 1.0)


def _conv2_pool(P, G, rinv, h, Wl, bl, Wr, batchp):
    return pl.pallas_call(
        _conv2_pool_kernel,
        grid=(NGRID,),
        in_specs=[
            pl.BlockSpec((1, NBLK, D), lambda i: (0, i, 0)),
            pl.BlockSpec((1, NBLK, D), lambda i: (1, i, 0)),
            pl.BlockSpec((NBLK, D), lambda i: (i, 0)),
            pl.BlockSpec((NBLK, 1), lambda i: (i, 0)),
            pl.BlockSpec((NBLK, D), lambda i: (i, 0)),
            pl.BlockSpec((D, D), lambda i: (0, 0)),
            pl.BlockSpec((1, D), lambda i: (0, 0)),
            pl.BlockSpec((D, D), lambda i: (0, 0)),
            pl.BlockSpec((NBLK, 1), lambda i: (i, 0)),
        ],
        out_specs=pl.BlockSpec((NG, D), lambda i: (0, 0)),
        out_shape=jax.ShapeDtypeStruct((NG, D), jnp.float32),
        scratch_shapes=[
            pltpu.VMEM((NG, D), jnp.float32),
            pltpu.VMEM((NG, D), jnp.float32),
        ],
    )(P, P, G, rinv, h, Wl, bl, Wr, batchp)


def kernel(x, edge_index, edge_attr, batch, Wn1, bn1, Wn2, bn2,
           We1, be1, We2, be2, conv_params):
    f32 = jnp.float32
    src = edge_index[0].astype(jnp.int32)
    dst = edge_index[1].astype(jnp.int32)

    xp = jnp.zeros((NP, D), f32).at[:N].set(x)
    eap = jnp.zeros((EP, DE), f32).at[:E].set(edge_attr)
    src2d = jnp.concatenate([src, jnp.zeros((EP - E,), jnp.int32)]).reshape(EP // CH, CH)
    dst2d = jnp.concatenate([dst, jnp.full((EP - E,), GARB, jnp.int32)]).reshape(EP // CH, CH)
    batchp = jnp.concatenate([batch.astype(jnp.int32),
                              jnp.full((NP - N,), NG, jnp.int32)]).reshape(NP, 1)
    z128 = jnp.zeros((CH, D), f32)
    one128 = jnp.ones((CH, D), f32)
    Wts = jnp.stack([p["Wt"] for p in conv_params])
    bts = jnp.stack([p["bt"] for p in conv_params]).reshape(3, 1, D)

    h = _node_mlp(xp, Wn1, bn1.reshape(1, D), Wn2, bn2.reshape(1, D))
    t = _edge_mlp(eap, We1, be1.reshape(1, D))
    Tp, Cp = _seg_t(t, dst2d, z128, one128)
    G0, G1, G2, rinv = _prep(Tp, Cp, We2, be2.reshape(1, D), Wts, bts)

    for i, (G, p) in enumerate(zip((G0, G1, G2), conv_params)):
        P = _seg_h(h, src2d, dst2d, z128)
        bl = p["bl"].reshape(1, D)
        if i < 2:
            h = _conv(P, G, rinv, h, p["Wl"], bl, p["Wr"])
        else:
            return _conv2_pool(P, G, rinv, h, p["Wl"], bl, p["Wr"], batchp)


# speedup vs baseline: 1.1677x; 1.1677x over previous
import jax
import jax.numpy as jnp
from jax import lax
from jax.experimental import pallas as pl
from jax.experimental.pallas import tpu as pltpu
from jax.experimental.pallas import tpu_sc as plsc

N = 10000
NP = 10240
GARB = 10016
D = 128
DE = 16
E = 320000
EP = 327680
CH = 128
CPT = 80
IST = 8
NC, NS = 2, 16
RPS = NP // NS
NBLK = 512
NGRID = NP // NBLK
NG = 64


def _mesh():
    return plsc.VectorSubcoreMesh(core_axis_name="c", subcore_axis_name="s")


def _seg_t_kernel(t_hbm, dst_hbm, z128_hbm, one128_hbm,
                  outT_hbm, outC_hbm,
                  dsti, rows0, rows1, acc, sg0, sg1, ss0, ss1):
    c = lax.axis_index("c")
    s = lax.axis_index("s")
    w = c * NS + s
    rows = (rows0, rows1)
    sg = (sg0, sg1)
    ss = (ss0, ss1)

    pltpu.sync_copy(z128_hbm, rows0)

    @pl.loop(0, RPS // CH)
    def _(k):
        pltpu.sync_copy(rows0, acc.at[pl.ds(s * RPS + k * CH, CH)])

    plsc.subcore_barrier()

    @pl.loop(0, CPT // IST)
    def _(jo):
        base = w * CPT + jo * IST
        pltpu.sync_copy(dst_hbm.at[pl.ds(base, IST)], dsti)
        gd = [None, None]
        sd = [None, None]
        gd[0] = pltpu.async_copy(t_hbm.at[pl.ds(base * CH, CH)], rows[0], sg[0])
        for jj in range(IST):
            k = jj % 2
            gd[k].wait()
            if jj + 1 < IST:
                k2 = (jj + 1) % 2
                if sd[k2] is not None:
                    sd[k2].wait()
                gd[k2] = pltpu.async_copy(
                    t_hbm.at[pl.ds((base + jj + 1) * CH, CH)], rows[k2], sg[k2])
            sd[k] = pltpu.async_copy(rows[k], acc.at[dsti.at[jj]], ss[k], add=True)
        sd[0].wait()
        sd[1].wait()

    plsc.subcore_barrier()

    @pl.loop(0, RPS // CH)
    def _(k):
        pltpu.sync_copy(acc.at[pl.ds(s * RPS + k * CH, CH)], rows0)
        pltpu.sync_copy(rows0, outT_hbm.at[c, pl.ds(s * RPS + k * CH, CH)])

    pltpu.sync_copy(z128_hbm, rows0)

    @pl.loop(0, RPS // CH)
    def _(k):
        pltpu.sync_copy(rows0, acc.at[pl.ds(s * RPS + k * CH, CH)])

    pltpu.sync_copy(one128_hbm, rows0)
    plsc.subcore_barrier()

    @pl.loop(0, CPT // IST)
    def _(jo):
        pltpu.sync_copy(dst_hbm.at[pl.ds(w * CPT + jo * IST, IST)], dsti)
        sd = []
        for jj in range(IST):
            sd.append(pltpu.async_copy(rows0, acc.at[dsti.at[jj]], ss0, add=True))
        for d in sd:
            d.wait()

    plsc.subcore_barrier()

    @pl.loop(0, RPS // CH)
    def _(k):
        pltpu.sync_copy(acc.at[pl.ds(s * RPS + k * CH, CH)], rows1)
        pltpu.sync_copy(rows1, outC_hbm.at[c, pl.ds(s * RPS + k * CH, CH)])


def _seg_t(t, dst2d, z128, one128):
    f = pl.kernel(
        _seg_t_kernel,
        out_type=(jax.ShapeDtypeStruct((NC, NP, D), jnp.float32),
                  jax.ShapeDtypeStruct((NC, NP, D), jnp.float32)),
        mesh=_mesh(),
        scratch_types=[
            pltpu.VMEM((IST, CH), jnp.int32),
            pltpu.VMEM((CH, D), jnp.float32),
            pltpu.VMEM((CH, D), jnp.float32),
            pltpu.VMEM_SHARED((NP, D), jnp.float32),
            pltpu.SemaphoreType.DMA,
            pltpu.SemaphoreType.DMA,
            pltpu.SemaphoreType.DMA,
            pltpu.SemaphoreType.DMA,
        ],
    )
    return f(t, dst2d, z128, one128)


CPT0 = 120
CPT1 = 2 * CPT - CPT0


def _seg_h_kernel(h_hbm, src_hbm, dst_hbm, z128_hbm,
                  out_hbm,
                  srci, dsti, rows0, rows1, acc, sg0, sg1, ss0, ss1):
    c = lax.axis_index("c")
    s = lax.axis_index("s")
    rows = (rows0, rows1)
    sg = (sg0, sg1)
    ss = (ss0, ss1)
    pltpu.sync_copy(z128_hbm, rows0)

    @pl.loop(0, RPS // CH)
    def _(k):
        pltpu.sync_copy(rows0, acc.at[pl.ds(s * RPS + k * CH, CH)])

    plsc.subcore_barrier()

    my_cpt = jnp.where(c == 0, CPT0, CPT1)
    my_base = jnp.where(c == 0, s * CPT0, NS * CPT0 + s * CPT1)

    @pl.loop(0, my_cpt // IST)
    def _(jo):
        pltpu.sync_copy(src_hbm.at[pl.ds(my_base + jo * IST, IST)], srci)
        pltpu.sync_copy(dst_hbm.at[pl.ds(my_base + jo * IST, IST)], dsti)
        gd = [None, None]
        sd = [None, None]
        gd[0] = pltpu.async_copy(h_hbm.at[srci.at[0]], rows[0], sg[0])
        for jj in range(IST):
            k = jj % 2
            gd[k].wait()
            if jj + 1 < IST:
                k2 = (jj + 1) % 2
                if sd[k2] is not None:
                    sd[k2].wait()
                gd[k2] = pltpu.async_copy(h_hbm.at[srci.at[jj + 1]], rows[k2], sg[k2])
            sd[k] = pltpu.async_copy(rows[k], acc.at[dsti.at[jj]], ss[k], add=True)
        sd[0].wait()
        sd[1].wait()

    plsc.subcore_barrier()

    @pl.loop(0, RPS // CH)
    def _(k):
        pltpu.sync_copy(acc.at[pl.ds(s * RPS + k * CH, CH)], rows0)
        pltpu.sync_copy(rows0, out_hbm.at[c, pl.ds(s * RPS + k * CH, CH)])


def _seg_h(h, src2d, dst2d, z128):
    f = pl.kernel(
        _seg_h_kernel,
        out_type=jax.ShapeDtypeStruct((NC, NP, D), jnp.float32),
        mesh=_mesh(),
        scratch_types=[
            pltpu.VMEM((IST, CH), jnp.int32),
            pltpu.VMEM((IST, CH), jnp.int32),
            pltpu.VMEM((CH, D), jnp.float32),
            pltpu.VMEM((CH, D), jnp.float32),
            pltpu.VMEM_SHARED((NP, D), jnp.float32),
            pltpu.SemaphoreType.DMA,
            pltpu.SemaphoreType.DMA,
            pltpu.SemaphoreType.DMA,
            pltpu.SemaphoreType.DMA,
        ],
    )
    return f(h, src2d, dst2d, z128)


def _node_mlp_kernel(x_ref, w1_ref, b1_ref, w2_ref, b2_ref, o_ref):
    a = jnp.dot(x_ref[...], w1_ref[...], preferred_element_type=jnp.float32)
    a = jax.nn.relu(a + b1_ref[...])
    o_ref[...] = jnp.dot(a, w2_ref[...], preferred_element_type=jnp.float32) + b2_ref[...]


def _node_mlp(xp, Wn1, bn1, Wn2, bn2):
    return pl.pallas_call(
        _node_mlp_kernel,
        grid=(NGRID,),
        in_specs=[
            pl.BlockSpec((NBLK, D), lambda i: (i, 0)),
            pl.BlockSpec((D, D), lambda i: (0, 0)),
            pl.BlockSpec((1, D), lambda i: (0, 0)),
            pl.BlockSpec((D, D), lambda i: (0, 0)),
            pl.BlockSpec((1, D), lambda i: (0, 0)),
        ],
        out_specs=pl.BlockSpec((NBLK, D), lambda i: (i, 0)),
        out_shape=jax.ShapeDtypeStruct((NP, D), jnp.float32),
    )(xp, Wn1, bn1, Wn2, bn2)


EBLK = 2048


def _edge_mlp_kernel(ea_ref, w1_ref, b1_ref, o_ref):
    a = jnp.dot(ea_ref[...], w1_ref[...], preferred_element_type=jnp.float32)
    o_ref[...] = jax.nn.relu(a + b1_ref[...])


def _edge_mlp(eap, We1, be1):
    return pl.pallas_call(
        _edge_mlp_kernel,
        grid=(EP // EBLK,),
        in_specs=[
            pl.BlockSpec((EBLK, DE), lambda i: (i, 0)),
            pl.BlockSpec((DE, D), lambda i: (0, 0)),
            pl.BlockSpec((1, D), lambda i: (0, 0)),
        ],
        out_specs=pl.BlockSpec((EBLK, D), lambda i: (i, 0)),
        out_shape=jax.ShapeDtypeStruct((EP, D), jnp.float32),
    )(eap, We1, be1)


def _prep_kernel(t0_ref, t1_ref, c0_ref, c1_ref, we2_ref, be2_ref,
                 wt_ref, bt_ref, g0_ref, g1_ref, g2_ref, rinv_ref):
    tsum = t0_ref[0] + t1_ref[0]
    cnt = (c0_ref[0] + c1_ref[0])[:, 0:1]
    rinv_ref[...] = 1.0 / jnp.maximum(cnt, 1.0)
    for i, g_ref in enumerate((g0_ref, g1_ref, g2_ref)):
        m = jnp.dot(we2_ref[...], wt_ref[i], preferred_element_type=jnp.float32)
        v = jnp.dot(be2_ref[...], wt_ref[i], preferred_element_type=jnp.float32) + bt_ref[i]
        g_ref[...] = jnp.dot(tsum, m, preferred_element_type=jnp.float32) + cnt * v


def _prep(Tp, Cp, We2, be2, Wts, bts):
    return pl.pallas_call(
        _prep_kernel,
        grid=(NGRID,),
        in_specs=[
            pl.BlockSpec((1, NBLK, D), lambda i: (0, i, 0)),
            pl.BlockSpec((1, NBLK, D), lambda i: (1, i, 0)),
            pl.BlockSpec((1, NBLK, D), lambda i: (0, i, 0)),
            pl.BlockSpec((1, NBLK, D), lambda i: (1, i, 0)),
            pl.BlockSpec((D, D), lambda i: (0, 0)),
            pl.BlockSpec((1, D), lambda i: (0, 0)),
            pl.BlockSpec((3, D, D), lambda i: (0, 0, 0)),
            pl.BlockSpec((3, 1, D), lambda i: (0, 0, 0)),
        ],
        out_specs=[
            pl.BlockSpec((NBLK, D), lambda i: (i, 0)),
            pl.BlockSpec((NBLK, D), lambda i: (i, 0)),
            pl.BlockSpec((NBLK, D), lambda i: (i, 0)),
            pl.BlockSpec((NBLK, 1), lambda i: (i, 0)),
        ],
        out_shape=[
            jax.ShapeDtypeStruct((NP, D), jnp.float32),
            jax.ShapeDtypeStruct((NP, D), jnp.float32),
            jax.ShapeDtypeStruct((NP, D), jnp.float32),
            jax.ShapeDtypeStruct((NP, 1), jnp.float32),
        ],
    )(Tp, Tp, Cp, Cp, We2, be2, Wts, bts)


def _conv_kernel(p0_ref, p1_ref, g_ref, rinv_ref, h_ref, wl_ref, bl_ref,
                 wr_ref, o_ref):
    s = p0_ref[0] + p1_ref[0]
    agg = (s + g_ref[...]) * rinv_ref[...]
    out = (jnp.dot(agg, wl_ref[...], preferred_element_type=jnp.float32)
           + bl_ref[...]
           + jnp.dot(h_ref[...], wr_ref[...], preferred_element_type=jnp.float32))
    o_ref[...] = jax.nn.relu(h_ref[...] + out)


def _conv(P, G, rinv, h, Wl, bl, Wr):
    return pl.pallas_call(
        _conv_kernel,
        grid=(NGRID,),
        in_specs=[
            pl.BlockSpec((1, NBLK, D), lambda i: (0, i, 0)),
            pl.BlockSpec((1, NBLK, D), lambda i: (1, i, 0)),
            pl.BlockSpec((NBLK, D), lambda i: (i, 0)),
            pl.BlockSpec((NBLK, 1), lambda i: (i, 0)),
            pl.BlockSpec((NBLK, D), lambda i: (i, 0)),
            pl.BlockSpec((D, D), lambda i: (0, 0)),
            pl.BlockSpec((1, D), lambda i: (0, 0)),
            pl.BlockSpec((D, D), lambda i: (0, 0)),
        ],
        out_specs=pl.BlockSpec((NBLK, D), lambda i: (i, 0)),
        out_shape=jax.ShapeDtypeStruct((NP, D), jnp.float32),
    )(P, P, G, rinv, h, Wl, bl, Wr)


def _conv2_pool_kernel(p0_ref, p1_ref, g_ref, rinv_ref, h_ref, wl_ref, bl_ref,
                       wr_ref, batch_ref, o_ref, accs, accc):
    i = pl.program_id(0)
    s = p0_ref[0] + p1_ref[0]
    agg = (s + g_ref[...]) * rinv_ref[...]
    out = (jnp.dot(agg, wl_ref[...], preferred_element_type=jnp.float32)
           + bl_ref[...]
           + jnp.dot(h_ref[...], wr_ref[...], preferred_element_type=jnp.float32))
    gid = lax.broadcasted_iota(jnp.int32, (NBLK, NG), 1)
    oh = (batch_ref[...] == gid).astype(jnp.float32)
    dn = (((0,), (0,)), ((), ()))
    psum = lax.dot_general(oh, out, dn, preferred_element_type=jnp.float32)
    pcnt = lax.dot_general(oh, jnp.ones((NBLK, D), jnp.float32), dn,
                           preferred_element_type=jnp.float32)

    @pl.when(i == 0)
    def _():
        accs[...] = psum
        accc[...] = pcnt

    @pl.when(i > 0)
    def _():
        accs[...] += psum
        accc[...] += pcnt

    @pl.when(i == NGRID - 1)
    def _():
        o_ref[...] = accs[...] / jnp.maximum(accc[...], 1.0)


def _conv2_pool(P, G, rinv, h, Wl, bl, Wr, batchp):
    return pl.pallas_call(
        _conv2_pool_kernel,
        grid=(NGRID,),
        in_specs=[
            pl.BlockSpec((1, NBLK, D), lambda i: (0, i, 0)),
            pl.BlockSpec((1, NBLK, D), lambda i: (1, i, 0)),
            pl.BlockSpec((NBLK, D), lambda i: (i, 0)),
            pl.BlockSpec((NBLK, 1), lambda i: (i, 0)),
            pl.BlockSpec((NBLK, D), lambda i: (i, 0)),
            pl.BlockSpec((D, D), lambda i: (0, 0)),
            pl.BlockSpec((1, D), lambda i: (0, 0)),
            pl.BlockSpec((D, D), lambda i: (0, 0)),
            pl.BlockSpec((NBLK, 1), lambda i: (i, 0)),
        ],
        out_specs=pl.BlockSpec((NG, D), lambda i: (0, 0)),
        out_shape=jax.ShapeDtypeStruct((NG, D), jnp.float32),
        scratch_shapes=[
            pltpu.VMEM((NG, D), jnp.float32),
            pltpu.VMEM((NG, D), jnp.float32),
        ],
    )(P, P, G, rinv, h, Wl, bl, Wr, batchp)


def kernel(x, edge_index, edge_attr, batch, Wn1, bn1, Wn2, bn2,
           We1, be1, We2, be2, conv_params):
    f32 = jnp.float32
    src = edge_index[0].astype(jnp.int32)
    dst = edge_index[1].astype(jnp.int32)

    xp = jnp.zeros((NP, D), f32).at[:N].set(x)
    eap = jnp.zeros((EP, DE), f32).at[:E].set(edge_attr)
    src2d = jnp.concatenate([src, jnp.zeros((EP - E,), jnp.int32)]).reshape(EP // CH, CH)
    dst2d = jnp.concatenate([dst, jnp.full((EP - E,), GARB, jnp.int32)]).reshape(EP // CH, CH)
    batchp = jnp.concatenate([batch.astype(jnp.int32),
                              jnp.full((NP - N,), NG, jnp.int32)]).reshape(NP, 1)
    z128 = jnp.zeros((CH, D), f32)
    one128 = jnp.ones((CH, D), f32)
    Wts = jnp.stack([p["Wt"] for p in conv_params])
    bts = jnp.stack([p["bt"] for p in conv_params]).reshape(3, 1, D)

    h = _node_mlp(xp, Wn1, bn1.reshape(1, D), Wn2, bn2.reshape(1, D))
    t = _edge_mlp(eap, We1, be1.reshape(1, D))
    Tp, Cp = _seg_t(t, dst2d, z128, one128)
    G0, G1, G2, rinv = _prep(Tp, Cp, We2, be2.reshape(1, D), Wts, bts)

    for i, (G, p) in enumerate(zip((G0, G1, G2), conv_params)):
        P = _seg_h(h, src2d, dst2d, z128)
        bl = p["bl"].reshape(1, D)
        if i < 2:
            h = _conv(P, G, rinv, h, p["Wl"], bl, p["Wr"])
        else:
            return _conv2_pool(P, G, rinv, h, p["Wl"], bl, p["Wr"], batchp)
